# Initial kernel scaffold; baseline (speedup 1.0000x reference)
import jax
import jax.numpy as jnp
from jax.experimental import pallas as pl


def kernel(anchors, bboxes, rpn_loc, rpn_score):
    raise NotImplementedError("write your pallas kernel here")


# speedup vs baseline: 20.6727x; 20.6727x over previous
import functools

import jax
import jax.numpy as jnp
from jax import lax
from jax.experimental import pallas as pl
from jax.experimental.pallas import tpu as pltpu

N_ANCHOR = 20000
N_PAD = 20480
N_GT = 64
BATCH = 4
POS_IOU = 0.7
NEG_IOU = 0.3


def _loss_body(anchors_ref, bbox_ref, loc_ref, score_ref, out_ref):
    f32 = jnp.float32
    ax1 = anchors_ref[0:1, :]
    ay1 = anchors_ref[1:2, :]
    ax2 = anchors_ref[2:3, :]
    ay2 = anchors_ref[3:4, :]
    bt = bbox_ref[0]
    bx1 = bt[0:1, :].reshape(N_GT, 1)
    by1 = bt[1:2, :].reshape(N_GT, 1)
    bx2 = bt[2:3, :].reshape(N_GT, 1)
    by2 = bt[3:4, :].reshape(N_GT, 1)

    tlx = jnp.maximum(ax1, bx1)
    tly = jnp.maximum(ay1, by1)
    brx = jnp.minimum(ax2, bx2)
    bry = jnp.minimum(ay2, by2)
    iw = jnp.maximum(brx - tlx, 0.0)
    ih = jnp.maximum(bry - tly, 0.0)
    inter = iw * ih
    area_a = (ax2 - ax1) * (ay2 - ay1)
    area_b = (bx2 - bx1) * (by2 - by1)
    iou = inter / (area_a + area_b - inter + 1e-9)

    i_iota = lax.broadcasted_iota(jnp.int32, (N_GT, N_PAD), 1)
    g_iota = lax.broadcasted_iota(jnp.int32, (N_GT, N_PAD), 0)

    max_iou = jnp.max(iou, axis=0, keepdims=True)
    argmax_g = jnp.min(jnp.where(iou == max_iou, g_iota, N_GT),
                       axis=0, keepdims=True)

    colmax = jnp.max(iou, axis=1, keepdims=True)
    col_argmax = jnp.min(jnp.where(iou == colmax, i_iota, N_PAD),
                         axis=1, keepdims=True)

    lane_valid = i_iota < N_ANCHOR
    scat = jnp.max(jnp.where((argmax_g == g_iota) & lane_valid, i_iota, -1),
                   axis=1, keepdims=True)
    gt_argmax = jnp.where(scat >= 0, scat, col_argmax)

    valid_lane = lax.broadcasted_iota(jnp.int32, (1, N_PAD), 1) < N_ANCHOR
    member = jnp.max(jnp.where(gt_argmax == i_iota, 1, 0),
                     axis=0, keepdims=True) > 0
    pos = (max_iou >= POS_IOU) | member
    neg = (max_iou < NEG_IOU) & valid_lane
    valid = pos | neg

    onehot = argmax_g == g_iota
    mx1 = jnp.max(jnp.where(onehot, bx1, -1e30), axis=0, keepdims=True)
    my1 = jnp.max(jnp.where(onehot, by1, -1e30), axis=0, keepdims=True)
    mx2 = jnp.max(jnp.where(onehot, bx2, -1e30), axis=0, keepdims=True)
    my2 = jnp.max(jnp.where(onehot, by2, -1e30), axis=0, keepdims=True)

    eps = jnp.finfo(f32).eps
    w = ax2 - ax1
    h = ay2 - ay1
    cx = ax1 + w * 0.5
    cy = ay1 + h * 0.5
    dw_ = mx2 - mx1
    dh_ = my2 - my1
    dcx = mx1 + dw_ * 0.5
    dcy = my1 + dh_ * 0.5
    w = jnp.maximum(w, eps)
    h = jnp.maximum(h, eps)
    tdx = (dcx - cx) / w
    tdy = (dcy - cy) / h
    tdw = jnp.log(dw_ / w)
    tdh = jnp.log(dh_ / h)

    lr = loc_ref[0]
    d0 = jnp.abs(tdx - lr[0:1, :])
    d1 = jnp.abs(tdy - lr[1:2, :])
    d2 = jnp.abs(tdw - lr[2:3, :])
    d3 = jnp.abs(tdh - lr[3:4, :])

    def sl1(d):
        return jnp.where(d < 1.0, 0.5 * d * d, d - 0.5)

    rl = sl1(d0) + sl1(d1) + sl1(d2) + sl1(d3)
    posf = pos.astype(f32)
    num_pos = jnp.maximum(jnp.sum(posf), 1.0)
    loc_loss = jnp.sum(rl * posf) / num_pos

    s0 = score_ref[0][0:1, :]
    s1 = score_ref[0][1:2, :]
    m = jnp.maximum(s0, s1)
    lse = m + jnp.log(jnp.exp(s0 - m) + jnp.exp(s1 - m))
    ce = lse - jnp.where(pos, s1, s0)
    validf = valid.astype(f32)
    num_valid = jnp.maximum(jnp.sum(validf), 1.0)
    cls_loss = jnp.sum(jnp.where(valid, ce, 0.0)) / num_valid

    out_ref[:, :, :] = (loc_loss + cls_loss).reshape(1, 1, 1)


@jax.jit
def kernel(anchors, bboxes, rpn_loc, rpn_score):
    f32 = jnp.float32
    pad = N_PAD - N_ANCHOR
    anchors_t = jnp.pad(anchors.astype(f32).T, ((0, 0), (0, pad)))
    bboxes_t = jnp.transpose(bboxes.astype(f32), (0, 2, 1))
    loc_t = jnp.pad(jnp.transpose(rpn_loc, (0, 2, 1)), ((0, 0), (0, 0), (0, pad)))
    score_t = jnp.pad(jnp.transpose(rpn_score, (0, 2, 1)), ((0, 0), (0, 0), (0, pad)))

    losses = pl.pallas_call(
        _loss_body,
        grid=(BATCH,),
        in_specs=[
            pl.BlockSpec((4, N_PAD), lambda b: (0, 0)),
            pl.BlockSpec((1, 4, N_GT), lambda b: (b, 0, 0)),
            pl.BlockSpec((1, 4, N_PAD), lambda b: (b, 0, 0)),
            pl.BlockSpec((1, 2, N_PAD), lambda b: (b, 0, 0)),
        ],
        out_specs=pl.BlockSpec((1, 1, 1), lambda b: (b, 0, 0)),
        out_shape=jax.ShapeDtypeStruct((BATCH, 1, 1), f32),
    )(anchors_t, bboxes_t, loc_t, score_t)
    return jnp.sum(losses)
